# baseline (speedup 1.0000x reference)
import functools

import jax
import jax.numpy as jnp
import numpy as np
from jax import lax
from jax.experimental import pallas as pl
from jax.experimental.pallas import tpu as pltpu
from jax.experimental.pallas import tpu_sc as plsc

_N = 10000
_E = 320000
_DIN = 128
_DOUT = 64
_NC = 2
_NS = 16
_NW = _NC * _NS
_B = 80
_EPW = _E // _NW
_CH = _EPW // _B
_NPAD = 10240
_RPW = _NPAD // _NS

_mesh = plsc.VectorSubcoreMesh(core_axis_name="c", subcore_axis_name="s")


@functools.partial(
    pl.kernel,
    out_type=[
        jax.ShapeDtypeStruct((_NC, _NPAD, _DIN), jnp.float32),
        jax.ShapeDtypeStruct((_NC, _NPAD, 8), jnp.float32),
    ],
    mesh=_mesh,
    compiler_params=pltpu.CompilerParams(use_tc_tiling_on_sc=False),
    scratch_types=[
        pltpu.VMEM((_CH, _B), jnp.int32),
        pltpu.VMEM((_CH, _B), jnp.int32),
        pltpu.VMEM((_B, _DIN), jnp.float32),
        pltpu.VMEM((_B, _DIN), jnp.float32),
        pltpu.VMEM((_B, 8), jnp.float32),
        pltpu.VMEM_SHARED((_NPAD, _DIN), jnp.float32),
        pltpu.VMEM_SHARED((_NPAD, 8), jnp.float32),
        pltpu.SemaphoreType.DMA,
        pltpu.SemaphoreType.DMA,
    ],
)
def _agg1c(x_hbm, src_hbm, dst_hbm, zd_hbm, zc_hbm, ones_hbm,
           p_out, c_out, src_v, dst_v, rows0, rows1, ones_v,
           acc_sh, cacc_sh, sem0, sem1):
    cid = lax.axis_index("c")
    sid = lax.axis_index("s")
    wid = cid * _NS + sid
    r0 = sid * _RPW
    pltpu.sync_copy(zd_hbm.at[pl.ds(r0, _RPW)], acc_sh.at[pl.ds(r0, _RPW)])
    pltpu.sync_copy(zc_hbm.at[pl.ds(r0, _RPW)], cacc_sh.at[pl.ds(r0, _RPW)])
    pltpu.sync_copy(src_hbm.at[wid], src_v)
    pltpu.sync_copy(dst_hbm.at[wid], dst_v)
    pltpu.sync_copy(ones_hbm, ones_v)
    plsc.subcore_barrier()

    pltpu.async_copy(x_hbm.at[src_v.at[0]], rows0, sem0)

    def body(j, carry):
        i0 = 2 * j
        pltpu.async_copy(x_hbm.at[src_v.at[i0 + 1]], rows1, sem1)
        pltpu.sync_copy(ones_v, cacc_sh.at[dst_v.at[i0]], add=True)
        pltpu.make_async_copy(x_hbm.at[src_v.at[i0]], rows0, sem0).wait()
        pltpu.sync_copy(rows0, acc_sh.at[dst_v.at[i0]], add=True)
        pltpu.async_copy(x_hbm.at[src_v.at[i0 + 2]], rows0, sem0)
        pltpu.sync_copy(ones_v, cacc_sh.at[dst_v.at[i0 + 1]], add=True)
        pltpu.make_async_copy(
            x_hbm.at[src_v.at[i0 + 1]], rows1, sem1).wait()
        pltpu.sync_copy(rows1, acc_sh.at[dst_v.at[i0 + 1]], add=True)
        return carry

    lax.fori_loop(0, (_CH - 1) // 2, body, 0)
    pltpu.sync_copy(ones_v, cacc_sh.at[dst_v.at[_CH - 1]], add=True)
    pltpu.make_async_copy(x_hbm.at[src_v.at[_CH - 1]], rows0, sem0).wait()
    pltpu.sync_copy(rows0, acc_sh.at[dst_v.at[_CH - 1]], add=True)
    plsc.subcore_barrier()
    pltpu.sync_copy(acc_sh.at[pl.ds(r0, _RPW)], p_out.at[cid, pl.ds(r0, _RPW)])
    pltpu.sync_copy(cacc_sh.at[pl.ds(r0, _RPW)], c_out.at[cid, pl.ds(r0, _RPW)])


_B2 = 128
_CH2 = 80
_EPW2 = _CH2 * _B2
_PAD2 = _EPW2 - _EPW


@functools.partial(
    pl.kernel,
    out_type=jax.ShapeDtypeStruct((_NC, _NPAD, _DOUT), jnp.float32),
    mesh=_mesh,
    compiler_params=pltpu.CompilerParams(use_tc_tiling_on_sc=False),
    scratch_types=[
        pltpu.VMEM((_CH2, _B2), jnp.int32),
        pltpu.VMEM((_CH2, _B2), jnp.int32),
        pltpu.VMEM((_B2, _DOUT), jnp.float32),
        pltpu.VMEM((_B2, _DOUT), jnp.float32),
        pltpu.VMEM_SHARED((_NPAD, _DOUT), jnp.float32),
        pltpu.SemaphoreType.DMA,
        pltpu.SemaphoreType.DMA,
    ],
)
def _agg2(x_hbm, src_hbm, dst_hbm, zd_hbm,
          p_out, src_v, dst_v, rows0, rows1, acc_sh, sem0, sem1):
    cid = lax.axis_index("c")
    sid = lax.axis_index("s")
    wid = cid * _NS + sid
    r0 = sid * _RPW
    pltpu.sync_copy(zd_hbm.at[pl.ds(r0, _RPW)], acc_sh.at[pl.ds(r0, _RPW)])
    pltpu.sync_copy(src_hbm.at[wid], src_v)
    pltpu.sync_copy(dst_hbm.at[wid], dst_v)
    plsc.subcore_barrier()

    pltpu.async_copy(x_hbm.at[src_v.at[0]], rows0, sem0)

    def body(j, carry):
        i0 = 2 * j
        pltpu.async_copy(x_hbm.at[src_v.at[i0 + 1]], rows1, sem1)
        pltpu.make_async_copy(x_hbm.at[src_v.at[i0]], rows0, sem0).wait()
        pltpu.sync_copy(rows0, acc_sh.at[dst_v.at[i0]], add=True)
        @pl.when(i0 + 2 < _CH2)
        def _():
            pltpu.async_copy(x_hbm.at[src_v.at[i0 + 2]], rows0, sem0)
        pltpu.make_async_copy(
            x_hbm.at[src_v.at[i0 + 1]], rows1, sem1).wait()
        pltpu.sync_copy(rows1, acc_sh.at[dst_v.at[i0 + 1]], add=True)
        return carry

    lax.fori_loop(0, _CH2 // 2, body, 0)
    plsc.subcore_barrier()
    pltpu.sync_copy(acc_sh.at[pl.ds(r0, _RPW)], p_out.at[cid, pl.ds(r0, _RPW)])


_BN = 1000


def _dense1_body(p_ref, c_ref, x_ref, wl_ref, b_ref, wr_ref, w2_ref,
                 h_ref, g_ref):
    cnt = jnp.sum(c_ref[...], axis=(0, 2))
    denom = jnp.maximum(cnt, 1.0)
    mean = (p_ref[0] + p_ref[1]) / denom[:, None]
    h = (jnp.dot(mean, wl_ref[...], preferred_element_type=jnp.float32)
         + b_ref[...]
         + jnp.dot(x_ref[...], wr_ref[...], preferred_element_type=jnp.float32))
    h = jnp.maximum(h, 0.0)
    h_ref[...] = h
    g_ref[...] = jnp.dot(h, w2_ref[...], preferred_element_type=jnp.float32)


def _dense1(P, C, x, WlT, b, WrT, W2T):
    return pl.pallas_call(
        _dense1_body,
        grid=(_N // _BN,),
        in_specs=[
            pl.BlockSpec((_NC, _BN, _DIN), lambda i: (0, i, 0)),
            pl.BlockSpec((_NC, _BN, 8), lambda i: (0, i, 0)),
            pl.BlockSpec((_BN, _DIN), lambda i: (i, 0)),
            pl.BlockSpec((_DIN, _DIN), lambda i: (0, 0)),
            pl.BlockSpec((1, _DIN), lambda i: (0, 0)),
            pl.BlockSpec((_DIN, _DIN), lambda i: (0, 0)),
            pl.BlockSpec((_DIN, _DOUT), lambda i: (0, 0)),
        ],
        out_specs=[
            pl.BlockSpec((_BN, _DIN), lambda i: (i, 0)),
            pl.BlockSpec((_BN, _DOUT), lambda i: (i, 0)),
        ],
        out_shape=[
            jax.ShapeDtypeStruct((_N, _DIN), jnp.float32),
            jax.ShapeDtypeStruct((_N, _DOUT), jnp.float32),
        ],
    )(P, C, x, WlT, b, WrT, W2T)


def _dense2_body(p_ref, c_ref, h_ref, w_ref, b_ref, o_ref):
    cnt = jnp.sum(c_ref[...], axis=(0, 2))
    denom = jnp.maximum(cnt, 1.0)
    mean = (p_ref[0] + p_ref[1]) / denom[:, None]
    o = (mean + b_ref[...]
         + jnp.dot(h_ref[...], w_ref[...], preferred_element_type=jnp.float32))
    m = jnp.max(o, axis=-1, keepdims=True)
    lse = jnp.log(jnp.sum(jnp.exp(o - m), axis=-1, keepdims=True)) + m
    o_ref[...] = o - lse


def _dense2(P, C, h, WrT, b):
    return pl.pallas_call(
        _dense2_body,
        grid=(_N // _BN,),
        in_specs=[
            pl.BlockSpec((_NC, _BN, _DOUT), lambda i: (0, i, 0)),
            pl.BlockSpec((_NC, _BN, 8), lambda i: (0, i, 0)),
            pl.BlockSpec((_BN, _DIN), lambda i: (i, 0)),
            pl.BlockSpec((_DIN, _DOUT), lambda i: (0, 0)),
            pl.BlockSpec((1, _DOUT), lambda i: (0, 0)),
        ],
        out_specs=pl.BlockSpec((_BN, _DOUT), lambda i: (i, 0)),
        out_shape=jax.ShapeDtypeStruct((_N, _DOUT), jnp.float32),
    )(P, C, h, WrT, b)


def kernel(x, edge_index, W1_l, b1_l, W1_r, W2_l, b2_l, W2_r):
    src = edge_index[0].reshape(_NW, _CH, _B)
    dst = edge_index[1].reshape(_NW, _CH, _B)
    pad_src = jnp.zeros((_NW, _PAD2), edge_index.dtype)
    pad_dst = jnp.broadcast_to(
        (_N + jnp.arange(_NW, dtype=edge_index.dtype))[:, None], (_NW, _PAD2))
    src2 = jnp.concatenate(
        [edge_index[0].reshape(_NW, _EPW), pad_src], axis=1
    ).reshape(_NW, _CH2, _B2)
    dst2 = jnp.concatenate(
        [edge_index[1].reshape(_NW, _EPW), pad_dst], axis=1
    ).reshape(_NW, _CH2, _B2)
    zd = jnp.zeros((_NPAD, _DIN), jnp.float32)
    zc = jnp.zeros((_NPAD, 8), jnp.float32)
    z64 = jnp.zeros((_NPAD, _DOUT), jnp.float32)
    ones = jnp.zeros((_B, 8), jnp.float32).at[:, 0].set(1.0)
    P1, C1 = _agg1c(x, src, dst, zd, zc, ones)
    h, g = _dense1(P1, C1, x, W1_l.T, b1_l.reshape(1, -1), W1_r.T, W2_l.T)
    P2 = _agg2(g, src2, dst2, z64)
    return _dense2(P2, C1, h, W2_r.T, b2_l.reshape(1, -1))


# speedup vs baseline: 1.4360x; 1.4360x over previous
import functools

import jax
import jax.numpy as jnp
import numpy as np
from jax import lax
from jax.experimental import pallas as pl
from jax.experimental.pallas import tpu as pltpu
from jax.experimental.pallas import tpu_sc as plsc

_N = 10000
_E = 320000
_DIN = 128
_DOUT = 64
_NC = 2
_NS = 16
_NW = _NC * _NS
_B = 80
_EPW = _E // _NW
_CH = _EPW // _B
_NPAD = 10240
_RPW = _NPAD // _NS

_mesh = plsc.VectorSubcoreMesh(core_axis_name="c", subcore_axis_name="s")


@functools.partial(
    pl.kernel,
    out_type=[
        jax.ShapeDtypeStruct((_NC, _NPAD, _DIN), jnp.float32),
        jax.ShapeDtypeStruct((_NC, _NPAD, 8), jnp.float32),
    ],
    mesh=_mesh,
    compiler_params=pltpu.CompilerParams(use_tc_tiling_on_sc=False),
    scratch_types=[
        pltpu.VMEM((_CH, _B), jnp.int32),
        pltpu.VMEM((_CH, _B), jnp.int32),
        pltpu.VMEM((_B, _DIN), jnp.float32),
        pltpu.VMEM((_B, _DIN), jnp.float32),
        pltpu.VMEM((_B, 8), jnp.float32),
        pltpu.VMEM_SHARED((_NPAD, _DIN), jnp.float32),
        pltpu.VMEM_SHARED((_NPAD, 8), jnp.float32),
        pltpu.SemaphoreType.DMA,
        pltpu.SemaphoreType.DMA,
    ],
)
def _agg1c(x_hbm, src_hbm, dst_hbm, zd_hbm, zc_hbm, ones_hbm,
           p_out, c_out, src_v, dst_v, rows0, rows1, ones_v,
           acc_sh, cacc_sh, sem0, sem1):
    cid = lax.axis_index("c")
    sid = lax.axis_index("s")
    wid = cid * _NS + sid
    r0 = sid * _RPW
    pltpu.sync_copy(zd_hbm.at[pl.ds(r0, _RPW)], acc_sh.at[pl.ds(r0, _RPW)])
    pltpu.sync_copy(zc_hbm.at[pl.ds(r0, _RPW)], cacc_sh.at[pl.ds(r0, _RPW)])
    pltpu.sync_copy(src_hbm.at[wid], src_v)
    pltpu.sync_copy(dst_hbm.at[wid], dst_v)
    pltpu.sync_copy(ones_hbm, ones_v)
    plsc.subcore_barrier()

    pltpu.async_copy(x_hbm.at[src_v.at[0]], rows0, sem0)

    def body(j, carry):
        i0 = 2 * j
        pltpu.async_copy(x_hbm.at[src_v.at[i0 + 1]], rows1, sem1)
        pltpu.sync_copy(ones_v, cacc_sh.at[dst_v.at[i0]], add=True)
        pltpu.make_async_copy(x_hbm.at[src_v.at[i0]], rows0, sem0).wait()
        pltpu.sync_copy(rows0, acc_sh.at[dst_v.at[i0]], add=True)
        pltpu.async_copy(x_hbm.at[src_v.at[i0 + 2]], rows0, sem0)
        pltpu.sync_copy(ones_v, cacc_sh.at[dst_v.at[i0 + 1]], add=True)
        pltpu.make_async_copy(
            x_hbm.at[src_v.at[i0 + 1]], rows1, sem1).wait()
        pltpu.sync_copy(rows1, acc_sh.at[dst_v.at[i0 + 1]], add=True)
        return carry

    lax.fori_loop(0, (_CH - 1) // 2, body, 0)
    pltpu.sync_copy(ones_v, cacc_sh.at[dst_v.at[_CH - 1]], add=True)
    pltpu.make_async_copy(x_hbm.at[src_v.at[_CH - 1]], rows0, sem0).wait()
    pltpu.sync_copy(rows0, acc_sh.at[dst_v.at[_CH - 1]], add=True)
    plsc.subcore_barrier()
    pltpu.sync_copy(acc_sh.at[pl.ds(r0, _RPW)], p_out.at[cid, pl.ds(r0, _RPW)])
    pltpu.sync_copy(cacc_sh.at[pl.ds(r0, _RPW)], c_out.at[cid, pl.ds(r0, _RPW)])


@functools.partial(
    pl.kernel,
    out_type=jax.ShapeDtypeStruct((_NC, _NPAD, _DOUT), jnp.float32),
    mesh=_mesh,
    compiler_params=pltpu.CompilerParams(use_tc_tiling_on_sc=False),
    scratch_types=[
        pltpu.VMEM((_CH, _B), jnp.int32),
        pltpu.VMEM((_CH, _B), jnp.int32),
        pltpu.VMEM((_B, _DOUT), jnp.float32),
        pltpu.VMEM((_B, _DOUT), jnp.float32),
        pltpu.VMEM_SHARED((_NPAD, _DOUT), jnp.float32),
        pltpu.SemaphoreType.DMA,
        pltpu.SemaphoreType.DMA,
    ],
)
def _agg2(x_hbm, src_hbm, dst_hbm, zd_hbm,
          p_out, src_v, dst_v, rows0, rows1, acc_sh, sem0, sem1):
    cid = lax.axis_index("c")
    sid = lax.axis_index("s")
    wid = cid * _NS + sid
    r0 = sid * _RPW
    pltpu.sync_copy(zd_hbm.at[pl.ds(r0, _RPW)], acc_sh.at[pl.ds(r0, _RPW)])
    pltpu.sync_copy(src_hbm.at[wid], src_v)
    pltpu.sync_copy(dst_hbm.at[wid], dst_v)
    plsc.subcore_barrier()

    pltpu.async_copy(x_hbm.at[src_v.at[0]], rows0, sem0)

    def body(j, carry):
        i0 = 2 * j
        pltpu.async_copy(x_hbm.at[src_v.at[i0 + 1]], rows1, sem1)
        pltpu.make_async_copy(x_hbm.at[src_v.at[i0]], rows0, sem0).wait()
        pltpu.sync_copy(rows0, acc_sh.at[dst_v.at[i0]], add=True)
        pltpu.async_copy(x_hbm.at[src_v.at[i0 + 2]], rows0, sem0)
        pltpu.make_async_copy(
            x_hbm.at[src_v.at[i0 + 1]], rows1, sem1).wait()
        pltpu.sync_copy(rows1, acc_sh.at[dst_v.at[i0 + 1]], add=True)
        return carry

    lax.fori_loop(0, (_CH - 1) // 2, body, 0)
    pltpu.make_async_copy(
        x_hbm.at[src_v.at[_CH - 1]], rows0, sem0).wait()
    pltpu.sync_copy(rows0, acc_sh.at[dst_v.at[_CH - 1]], add=True)
    plsc.subcore_barrier()
    pltpu.sync_copy(acc_sh.at[pl.ds(r0, _RPW)], p_out.at[cid, pl.ds(r0, _RPW)])


_BN = 1000


def _dense1_body(p_ref, c_ref, x_ref, wl_ref, b_ref, wr_ref, w2_ref,
                 h_ref, g_ref):
    cnt = jnp.sum(c_ref[...], axis=(0, 2))
    denom = jnp.maximum(cnt, 1.0)
    mean = (p_ref[0] + p_ref[1]) / denom[:, None]
    h = (jnp.dot(mean, wl_ref[...], preferred_element_type=jnp.float32)
         + b_ref[...]
         + jnp.dot(x_ref[...], wr_ref[...], preferred_element_type=jnp.float32))
    h = jnp.maximum(h, 0.0)
    h_ref[...] = h
    g_ref[...] = jnp.dot(h, w2_ref[...], preferred_element_type=jnp.float32)


def _dense1(P, C, x, WlT, b, WrT, W2T):
    return pl.pallas_call(
        _dense1_body,
        grid=(_N // _BN,),
        in_specs=[
            pl.BlockSpec((_NC, _BN, _DIN), lambda i: (0, i, 0)),
            pl.BlockSpec((_NC, _BN, 8), lambda i: (0, i, 0)),
            pl.BlockSpec((_BN, _DIN), lambda i: (i, 0)),
            pl.BlockSpec((_DIN, _DIN), lambda i: (0, 0)),
            pl.BlockSpec((1, _DIN), lambda i: (0, 0)),
            pl.BlockSpec((_DIN, _DIN), lambda i: (0, 0)),
            pl.BlockSpec((_DIN, _DOUT), lambda i: (0, 0)),
        ],
        out_specs=[
            pl.BlockSpec((_BN, _DIN), lambda i: (i, 0)),
            pl.BlockSpec((_BN, _DOUT), lambda i: (i, 0)),
        ],
        out_shape=[
            jax.ShapeDtypeStruct((_N, _DIN), jnp.float32),
            jax.ShapeDtypeStruct((_N, _DOUT), jnp.float32),
        ],
    )(P, C, x, WlT, b, WrT, W2T)


def _dense2_body(p_ref, c_ref, h_ref, w_ref, b_ref, o_ref):
    cnt = jnp.sum(c_ref[...], axis=(0, 2))
    denom = jnp.maximum(cnt, 1.0)
    mean = (p_ref[0] + p_ref[1]) / denom[:, None]
    o = (mean + b_ref[...]
         + jnp.dot(h_ref[...], w_ref[...], preferred_element_type=jnp.float32))
    m = jnp.max(o, axis=-1, keepdims=True)
    lse = jnp.log(jnp.sum(jnp.exp(o - m), axis=-1, keepdims=True)) + m
    o_ref[...] = o - lse


def _dense2(P, C, h, WrT, b):
    return pl.pallas_call(
        _dense2_body,
        grid=(_N // _BN,),
        in_specs=[
            pl.BlockSpec((_NC, _BN, _DOUT), lambda i: (0, i, 0)),
            pl.BlockSpec((_NC, _BN, 8), lambda i: (0, i, 0)),
            pl.BlockSpec((_BN, _DIN), lambda i: (i, 0)),
            pl.BlockSpec((_DIN, _DOUT), lambda i: (0, 0)),
            pl.BlockSpec((1, _DOUT), lambda i: (0, 0)),
        ],
        out_specs=pl.BlockSpec((_BN, _DOUT), lambda i: (i, 0)),
        out_shape=jax.ShapeDtypeStruct((_N, _DOUT), jnp.float32),
    )(P, C, h, WrT, b)


def kernel(x, edge_index, W1_l, b1_l, W1_r, W2_l, b2_l, W2_r):
    src = edge_index[0].reshape(_NW, _CH, _B)
    dst = edge_index[1].reshape(_NW, _CH, _B)
    zd = jnp.zeros((_NPAD, _DIN), jnp.float32)
    zc = jnp.zeros((_NPAD, 8), jnp.float32)
    z64 = jnp.zeros((_NPAD, _DOUT), jnp.float32)
    ones = jnp.zeros((_B, 8), jnp.float32).at[:, 0].set(1.0)
    P1, C1 = _agg1c(x, src, dst, zd, zc, ones)
    h, g = _dense1(P1, C1, x, W1_l.T, b1_l.reshape(1, -1), W1_r.T, W2_l.T)
    P2 = _agg2(g, src, dst, z64)
    return _dense2(P2, C1, h, W2_r.T, b2_l.reshape(1, -1))


# speedup vs baseline: 1.5430x; 1.0746x over previous
import functools

import jax
import jax.numpy as jnp
import numpy as np
from jax import lax
from jax.experimental import pallas as pl
from jax.experimental.pallas import tpu as pltpu
from jax.experimental.pallas import tpu_sc as plsc

_N = 10000
_E = 320000
_DIN = 128
_DOUT = 64
_NC = 2
_NS = 16
_NW = _NC * _NS
_B = 80
_EPW = _E // _NW
_CH = _EPW // _B
_NPAD = 10240
_RPW = _NPAD // _NS

_mesh = plsc.VectorSubcoreMesh(core_axis_name="c", subcore_axis_name="s")


@functools.partial(
    pl.kernel,
    out_type=[
        jax.ShapeDtypeStruct((_NC, _NPAD, _DIN), jnp.float32),
        jax.ShapeDtypeStruct((_NC, _NPAD, 8), jnp.float32),
    ],
    mesh=_mesh,
    compiler_params=pltpu.CompilerParams(use_tc_tiling_on_sc=False),
    scratch_types=[
        pltpu.VMEM((_CH, _B), jnp.int32),
        pltpu.VMEM((_CH, _B), jnp.int32),
        pltpu.VMEM((_B, _DIN), jnp.float32),
        pltpu.VMEM((_B, _DIN), jnp.float32),
        pltpu.VMEM((_B, 8), jnp.float32),
        pltpu.VMEM_SHARED((_NPAD, _DIN), jnp.float32),
        pltpu.VMEM_SHARED((_NPAD, 8), jnp.float32),
        pltpu.SemaphoreType.DMA,
        pltpu.SemaphoreType.DMA,
    ],
)
def _agg1c(x_hbm, src_hbm, dst_hbm, zd_hbm, zc_hbm, ones_hbm,
           p_out, c_out, src_v, dst_v, rows0, rows1, ones_v,
           acc_sh, cacc_sh, sem0, sem1):
    cid = lax.axis_index("c")
    sid = lax.axis_index("s")
    wid = cid * _NS + sid
    r0 = sid * _RPW
    pltpu.sync_copy(zd_hbm.at[pl.ds(r0, _RPW)], acc_sh.at[pl.ds(r0, _RPW)])
    pltpu.sync_copy(zc_hbm.at[pl.ds(r0, _RPW)], cacc_sh.at[pl.ds(r0, _RPW)])
    pltpu.sync_copy(src_hbm.at[wid], src_v)
    pltpu.sync_copy(dst_hbm.at[wid], dst_v)
    pltpu.sync_copy(ones_hbm, ones_v)
    plsc.subcore_barrier()

    pltpu.async_copy(x_hbm.at[src_v.at[0]], rows0, sem0)

    def body(j, carry):
        i0 = 2 * j
        pltpu.async_copy(x_hbm.at[src_v.at[i0 + 1]], rows1, sem1)
        pltpu.sync_copy(ones_v, cacc_sh.at[dst_v.at[i0]], add=True)
        pltpu.make_async_copy(x_hbm.at[src_v.at[i0]], rows0, sem0).wait()
        pltpu.sync_copy(rows0, acc_sh.at[dst_v.at[i0]], add=True)
        pltpu.async_copy(x_hbm.at[src_v.at[i0 + 2]], rows0, sem0)
        pltpu.sync_copy(ones_v, cacc_sh.at[dst_v.at[i0 + 1]], add=True)
        pltpu.make_async_copy(
            x_hbm.at[src_v.at[i0 + 1]], rows1, sem1).wait()
        pltpu.sync_copy(rows1, acc_sh.at[dst_v.at[i0 + 1]], add=True)
        return carry

    lax.fori_loop(0, (_CH - 1) // 2, body, 0)
    pltpu.sync_copy(ones_v, cacc_sh.at[dst_v.at[_CH - 1]], add=True)
    pltpu.make_async_copy(x_hbm.at[src_v.at[_CH - 1]], rows0, sem0).wait()
    pltpu.sync_copy(rows0, acc_sh.at[dst_v.at[_CH - 1]], add=True)
    plsc.subcore_barrier()
    pltpu.sync_copy(acc_sh.at[pl.ds(r0, _RPW)], p_out.at[cid, pl.ds(r0, _RPW)])
    pltpu.sync_copy(cacc_sh.at[pl.ds(r0, _RPW)], c_out.at[cid, pl.ds(r0, _RPW)])


@functools.partial(
    pl.kernel,
    out_type=jax.ShapeDtypeStruct((_NC, _NPAD, _DOUT), jnp.float32),
    mesh=_mesh,
    compiler_params=pltpu.CompilerParams(use_tc_tiling_on_sc=False),
    scratch_types=[
        pltpu.VMEM((_CH, _B), jnp.int32),
        pltpu.VMEM((_CH, _B), jnp.int32),
        pltpu.VMEM((_B, _DOUT), jnp.float32),
        pltpu.VMEM((_B, _DOUT), jnp.float32),
        pltpu.VMEM((_B, _DOUT), jnp.float32),
        pltpu.VMEM_SHARED((_NPAD, _DOUT), jnp.float32),
        pltpu.SemaphoreType.DMA,
        pltpu.SemaphoreType.DMA,
        pltpu.SemaphoreType.DMA,
    ],
)
def _agg2(x_hbm, src_hbm, dst_hbm, zd_hbm,
          p_out, src_v, dst_v, rows0, rows1, rows2, acc_sh,
          sem0, sem1, sem2):
    cid = lax.axis_index("c")
    sid = lax.axis_index("s")
    wid = cid * _NS + sid
    r0 = sid * _RPW
    pltpu.sync_copy(zd_hbm.at[pl.ds(r0, _RPW)], acc_sh.at[pl.ds(r0, _RPW)])
    pltpu.sync_copy(src_hbm.at[wid], src_v)
    pltpu.sync_copy(dst_hbm.at[wid], dst_v)
    plsc.subcore_barrier()

    rows = (rows0, rows1, rows2)
    sems = (sem0, sem1, sem2)

    pltpu.async_copy(x_hbm.at[src_v.at[0]], rows0, sem0)
    pltpu.async_copy(x_hbm.at[src_v.at[1]], rows1, sem1)

    def body(j, carry):
        i0 = 3 * j
        for b in range(3):
            nb = (b + 2) % 3
            pltpu.async_copy(x_hbm.at[src_v.at[i0 + b + 2]], rows[nb], sems[nb])
            pltpu.make_async_copy(
                x_hbm.at[src_v.at[i0 + b]], rows[b], sems[b]).wait()
            pltpu.sync_copy(rows[b], acc_sh.at[dst_v.at[i0 + b]], add=True)
        return carry

    lax.fori_loop(0, (_CH - 2) // 3, body, 0)
    for i in (_CH - 2, _CH - 1):
        b = i % 3
        pltpu.make_async_copy(x_hbm.at[src_v.at[i]], rows[b], sems[b]).wait()
        pltpu.sync_copy(rows[b], acc_sh.at[dst_v.at[i]], add=True)
    plsc.subcore_barrier()
    pltpu.sync_copy(acc_sh.at[pl.ds(r0, _RPW)], p_out.at[cid, pl.ds(r0, _RPW)])


_BN = 1000


def _dense1_body(p_ref, c_ref, x_ref, wl_ref, b_ref, wr_ref, w2_ref,
                 h_ref, g_ref):
    cnt = jnp.sum(c_ref[...], axis=(0, 2))
    denom = jnp.maximum(cnt, 1.0)
    mean = (p_ref[0] + p_ref[1]) / denom[:, None]
    h = (jnp.dot(mean, wl_ref[...], preferred_element_type=jnp.float32)
         + b_ref[...]
         + jnp.dot(x_ref[...], wr_ref[...], preferred_element_type=jnp.float32))
    h = jnp.maximum(h, 0.0)
    h_ref[...] = h
    g_ref[...] = jnp.dot(h, w2_ref[...], preferred_element_type=jnp.float32)


def _dense1(P, C, x, WlT, b, WrT, W2T):
    return pl.pallas_call(
        _dense1_body,
        grid=(_N // _BN,),
        in_specs=[
            pl.BlockSpec((_NC, _BN, _DIN), lambda i: (0, i, 0)),
            pl.BlockSpec((_NC, _BN, 8), lambda i: (0, i, 0)),
            pl.BlockSpec((_BN, _DIN), lambda i: (i, 0)),
            pl.BlockSpec((_DIN, _DIN), lambda i: (0, 0)),
            pl.BlockSpec((1, _DIN), lambda i: (0, 0)),
            pl.BlockSpec((_DIN, _DIN), lambda i: (0, 0)),
            pl.BlockSpec((_DIN, _DOUT), lambda i: (0, 0)),
        ],
        out_specs=[
            pl.BlockSpec((_BN, _DIN), lambda i: (i, 0)),
            pl.BlockSpec((_BN, _DOUT), lambda i: (i, 0)),
        ],
        out_shape=[
            jax.ShapeDtypeStruct((_N, _DIN), jnp.float32),
            jax.ShapeDtypeStruct((_N, _DOUT), jnp.float32),
        ],
    )(P, C, x, WlT, b, WrT, W2T)


def _dense2_body(p_ref, c_ref, h_ref, w_ref, b_ref, o_ref):
    cnt = jnp.sum(c_ref[...], axis=(0, 2))
    denom = jnp.maximum(cnt, 1.0)
    mean = (p_ref[0] + p_ref[1]) / denom[:, None]
    o = (mean + b_ref[...]
         + jnp.dot(h_ref[...], w_ref[...], preferred_element_type=jnp.float32))
    m = jnp.max(o, axis=-1, keepdims=True)
    lse = jnp.log(jnp.sum(jnp.exp(o - m), axis=-1, keepdims=True)) + m
    o_ref[...] = o - lse


def _dense2(P, C, h, WrT, b):
    return pl.pallas_call(
        _dense2_body,
        grid=(_N // _BN,),
        in_specs=[
            pl.BlockSpec((_NC, _BN, _DOUT), lambda i: (0, i, 0)),
            pl.BlockSpec((_NC, _BN, 8), lambda i: (0, i, 0)),
            pl.BlockSpec((_BN, _DIN), lambda i: (i, 0)),
            pl.BlockSpec((_DIN, _DOUT), lambda i: (0, 0)),
            pl.BlockSpec((1, _DOUT), lambda i: (0, 0)),
        ],
        out_specs=pl.BlockSpec((_BN, _DOUT), lambda i: (i, 0)),
        out_shape=jax.ShapeDtypeStruct((_N, _DOUT), jnp.float32),
    )(P, C, h, WrT, b)


def kernel(x, edge_index, W1_l, b1_l, W1_r, W2_l, b2_l, W2_r):
    src = edge_index[0].reshape(_NW, _CH, _B)
    dst = edge_index[1].reshape(_NW, _CH, _B)
    zd = jnp.zeros((_NPAD, _DIN), jnp.float32)
    zc = jnp.zeros((_NPAD, 8), jnp.float32)
    z64 = jnp.zeros((_NPAD, _DOUT), jnp.float32)
    ones = jnp.zeros((_B, 8), jnp.float32).at[:, 0].set(1.0)
    P1, C1 = _agg1c(x, src, dst, zd, zc, ones)
    h, g = _dense1(P1, C1, x, W1_l.T, b1_l.reshape(1, -1), W1_r.T, W2_l.T)
    P2 = _agg2(g, src, dst, z64)
    return _dense2(P2, C1, h, W2_r.T, b2_l.reshape(1, -1))


# speedup vs baseline: 1.5837x; 1.0264x over previous
import functools

import jax
import jax.numpy as jnp
import numpy as np
from jax import lax
from jax.experimental import pallas as pl
from jax.experimental.pallas import tpu as pltpu
from jax.experimental.pallas import tpu_sc as plsc

_N = 10000
_E = 320000
_DIN = 128
_DOUT = 64
_NC = 2
_NS = 16
_NW = _NC * _NS
_B = 80
_EPW = _E // _NW
_CH = _EPW // _B
_NPAD = 10240
_RPW = _NPAD // _NS

_mesh = plsc.VectorSubcoreMesh(core_axis_name="c", subcore_axis_name="s")


@functools.partial(
    pl.kernel,
    out_type=[
        jax.ShapeDtypeStruct((_NC, _NPAD, _DIN), jnp.float32),
        jax.ShapeDtypeStruct((_NC, _NPAD, 8), jnp.float32),
    ],
    mesh=_mesh,
    compiler_params=pltpu.CompilerParams(use_tc_tiling_on_sc=False),
    scratch_types=[
        pltpu.VMEM((_CH, _B), jnp.int32),
        pltpu.VMEM((_CH, _B), jnp.int32),
        pltpu.VMEM((_B, _DIN), jnp.float32),
        pltpu.VMEM((_B, _DIN), jnp.float32),
        pltpu.VMEM((_B, 8), jnp.float32),
        pltpu.VMEM_SHARED((_NPAD, _DIN), jnp.float32),
        pltpu.VMEM_SHARED((_NPAD, 8), jnp.float32),
        pltpu.SemaphoreType.DMA,
        pltpu.SemaphoreType.DMA,
    ],
)
def _agg1c(x_hbm, src_hbm, dst_hbm, zd_hbm, zc_hbm, ones_hbm,
           p_out, c_out, src_v, dst_v, rows0, rows1, ones_v,
           acc_sh, cacc_sh, sem0, sem1):
    cid = lax.axis_index("c")
    sid = lax.axis_index("s")
    wid = cid * _NS + sid
    r0 = sid * _RPW
    pltpu.sync_copy(zd_hbm.at[pl.ds(r0, _RPW)], acc_sh.at[pl.ds(r0, _RPW)])
    pltpu.sync_copy(zc_hbm.at[pl.ds(r0, _RPW)], cacc_sh.at[pl.ds(r0, _RPW)])
    pltpu.sync_copy(src_hbm.at[wid], src_v)
    pltpu.sync_copy(dst_hbm.at[wid], dst_v)
    pltpu.sync_copy(ones_hbm, ones_v)
    plsc.subcore_barrier()

    pltpu.async_copy(x_hbm.at[src_v.at[0]], rows0, sem0)

    def body(j, carry):
        i0 = 2 * j
        pltpu.async_copy(x_hbm.at[src_v.at[i0 + 1]], rows1, sem1)
        pltpu.sync_copy(ones_v, cacc_sh.at[dst_v.at[i0]], add=True)
        pltpu.make_async_copy(x_hbm.at[src_v.at[i0]], rows0, sem0).wait()
        pltpu.sync_copy(rows0, acc_sh.at[dst_v.at[i0]], add=True)
        pltpu.async_copy(x_hbm.at[src_v.at[i0 + 2]], rows0, sem0)
        pltpu.sync_copy(ones_v, cacc_sh.at[dst_v.at[i0 + 1]], add=True)
        pltpu.make_async_copy(
            x_hbm.at[src_v.at[i0 + 1]], rows1, sem1).wait()
        pltpu.sync_copy(rows1, acc_sh.at[dst_v.at[i0 + 1]], add=True)
        return carry

    lax.fori_loop(0, (_CH - 1) // 2, body, 0)
    pltpu.sync_copy(ones_v, cacc_sh.at[dst_v.at[_CH - 1]], add=True)
    pltpu.make_async_copy(x_hbm.at[src_v.at[_CH - 1]], rows0, sem0).wait()
    pltpu.sync_copy(rows0, acc_sh.at[dst_v.at[_CH - 1]], add=True)
    plsc.subcore_barrier()
    pltpu.sync_copy(acc_sh.at[pl.ds(r0, _RPW)], p_out.at[cid, pl.ds(r0, _RPW)])
    pltpu.sync_copy(cacc_sh.at[pl.ds(r0, _RPW)], c_out.at[cid, pl.ds(r0, _RPW)])


@functools.partial(
    pl.kernel,
    out_type=jax.ShapeDtypeStruct((_NC, _NPAD, _DOUT), jnp.float32),
    mesh=_mesh,
    compiler_params=pltpu.CompilerParams(use_tc_tiling_on_sc=False),
    scratch_types=[
        pltpu.VMEM((_CH, _B), jnp.int32),
        pltpu.VMEM((_CH, _B), jnp.int32),
        pltpu.VMEM((_B, _DOUT), jnp.float32),
        pltpu.VMEM((_B, _DOUT), jnp.float32),
        pltpu.VMEM((_B, _DOUT), jnp.float32),
        pltpu.VMEM((_B, _DOUT), jnp.float32),
        pltpu.VMEM_SHARED((_NPAD, _DOUT), jnp.float32),
        pltpu.SemaphoreType.DMA,
        pltpu.SemaphoreType.DMA,
        pltpu.SemaphoreType.DMA,
        pltpu.SemaphoreType.DMA,
    ],
)
def _agg2(x_hbm, src_hbm, dst_hbm, zd_hbm,
          p_out, src_v, dst_v, rows0, rows1, rows2, rows3, acc_sh,
          sem0, sem1, sem2, sem3):
    cid = lax.axis_index("c")
    sid = lax.axis_index("s")
    wid = cid * _NS + sid
    r0 = sid * _RPW
    pltpu.sync_copy(zd_hbm.at[pl.ds(r0, _RPW)], acc_sh.at[pl.ds(r0, _RPW)])
    pltpu.sync_copy(src_hbm.at[wid], src_v)
    pltpu.sync_copy(dst_hbm.at[wid], dst_v)
    plsc.subcore_barrier()

    rows = (rows0, rows1, rows2, rows3)
    sems = (sem0, sem1, sem2, sem3)

    for i in range(3):
        pltpu.async_copy(x_hbm.at[src_v.at[i]], rows[i], sems[i])

    def body(j, carry):
        i0 = 4 * j
        for b in range(4):
            nb = (b + 3) % 4
            pltpu.async_copy(x_hbm.at[src_v.at[i0 + b + 3]], rows[nb], sems[nb])
            pltpu.make_async_copy(
                x_hbm.at[src_v.at[i0 + b]], rows[b], sems[b]).wait()
            pltpu.sync_copy(rows[b], acc_sh.at[dst_v.at[i0 + b]], add=True)
        return carry

    lax.fori_loop(0, (_CH - 5) // 4, body, 0)
    for i in range(_CH - 5, _CH):
        b = i % 4
        if i + 3 < _CH:
            nb = (i + 3) % 4
            pltpu.async_copy(
                x_hbm.at[src_v.at[i + 3]], rows[nb], sems[nb])
        pltpu.make_async_copy(x_hbm.at[src_v.at[i]], rows[b], sems[b]).wait()
        pltpu.sync_copy(rows[b], acc_sh.at[dst_v.at[i]], add=True)
    plsc.subcore_barrier()
    pltpu.sync_copy(acc_sh.at[pl.ds(r0, _RPW)], p_out.at[cid, pl.ds(r0, _RPW)])


_BN = 1000


def _dense1_body(p_ref, c_ref, x_ref, wl_ref, b_ref, wr_ref, w2_ref,
                 h_ref, g_ref):
    cnt = jnp.sum(c_ref[...], axis=(0, 2))
    denom = jnp.maximum(cnt, 1.0)
    mean = (p_ref[0] + p_ref[1]) / denom[:, None]
    h = (jnp.dot(mean, wl_ref[...], preferred_element_type=jnp.float32)
         + b_ref[...]
         + jnp.dot(x_ref[...], wr_ref[...], preferred_element_type=jnp.float32))
    h = jnp.maximum(h, 0.0)
    h_ref[...] = h
    g_ref[...] = jnp.dot(h, w2_ref[...], preferred_element_type=jnp.float32)


def _dense1(P, C, x, WlT, b, WrT, W2T):
    return pl.pallas_call(
        _dense1_body,
        grid=(_N // _BN,),
        in_specs=[
            pl.BlockSpec((_NC, _BN, _DIN), lambda i: (0, i, 0)),
            pl.BlockSpec((_NC, _BN, 8), lambda i: (0, i, 0)),
            pl.BlockSpec((_BN, _DIN), lambda i: (i, 0)),
            pl.BlockSpec((_DIN, _DIN), lambda i: (0, 0)),
            pl.BlockSpec((1, _DIN), lambda i: (0, 0)),
            pl.BlockSpec((_DIN, _DIN), lambda i: (0, 0)),
            pl.BlockSpec((_DIN, _DOUT), lambda i: (0, 0)),
        ],
        out_specs=[
            pl.BlockSpec((_BN, _DIN), lambda i: (i, 0)),
            pl.BlockSpec((_BN, _DOUT), lambda i: (i, 0)),
        ],
        out_shape=[
            jax.ShapeDtypeStruct((_N, _DIN), jnp.float32),
            jax.ShapeDtypeStruct((_N, _DOUT), jnp.float32),
        ],
    )(P, C, x, WlT, b, WrT, W2T)


def _dense2_body(p_ref, c_ref, h_ref, w_ref, b_ref, o_ref):
    cnt = jnp.sum(c_ref[...], axis=(0, 2))
    denom = jnp.maximum(cnt, 1.0)
    mean = (p_ref[0] + p_ref[1]) / denom[:, None]
    o = (mean + b_ref[...]
         + jnp.dot(h_ref[...], w_ref[...], preferred_element_type=jnp.float32))
    m = jnp.max(o, axis=-1, keepdims=True)
    lse = jnp.log(jnp.sum(jnp.exp(o - m), axis=-1, keepdims=True)) + m
    o_ref[...] = o - lse


def _dense2(P, C, h, WrT, b):
    return pl.pallas_call(
        _dense2_body,
        grid=(_N // _BN,),
        in_specs=[
            pl.BlockSpec((_NC, _BN, _DOUT), lambda i: (0, i, 0)),
            pl.BlockSpec((_NC, _BN, 8), lambda i: (0, i, 0)),
            pl.BlockSpec((_BN, _DIN), lambda i: (i, 0)),
            pl.BlockSpec((_DIN, _DOUT), lambda i: (0, 0)),
            pl.BlockSpec((1, _DOUT), lambda i: (0, 0)),
        ],
        out_specs=pl.BlockSpec((_BN, _DOUT), lambda i: (i, 0)),
        out_shape=jax.ShapeDtypeStruct((_N, _DOUT), jnp.float32),
    )(P, C, h, WrT, b)


def kernel(x, edge_index, W1_l, b1_l, W1_r, W2_l, b2_l, W2_r):
    src = edge_index[0].reshape(_NW, _CH, _B)
    dst = edge_index[1].reshape(_NW, _CH, _B)
    zd = jnp.zeros((_NPAD, _DIN), jnp.float32)
    zc = jnp.zeros((_NPAD, 8), jnp.float32)
    z64 = jnp.zeros((_NPAD, _DOUT), jnp.float32)
    ones = jnp.zeros((_B, 8), jnp.float32).at[:, 0].set(1.0)
    P1, C1 = _agg1c(x, src, dst, zd, zc, ones)
    h, g = _dense1(P1, C1, x, W1_l.T, b1_l.reshape(1, -1), W1_r.T, W2_l.T)
    P2 = _agg2(g, src, dst, z64)
    return _dense2(P2, C1, h, W2_r.T, b2_l.reshape(1, -1))


# speedup vs baseline: 1.6969x; 1.0715x over previous
import functools

import jax
import jax.numpy as jnp
from jax import lax
from jax.experimental import pallas as pl
from jax.experimental.pallas import tpu as pltpu
from jax.experimental.pallas import tpu_sc as plsc

_N = 10000
_E = 320000
_DIN = 128
_DOUT = 64
_NC = 2
_NS = 16
_NW = _NC * _NS
_B = 80
_EPW = _E // _NW
_CH = _EPW // _B
_NPAD = 10240
_RPW = _NPAD // _NS

_mesh = plsc.VectorSubcoreMesh(core_axis_name="c", subcore_axis_name="s")


_CHA = 63
_CHB = _CH - _CHA


@functools.partial(
    pl.kernel,
    out_type=[
        jax.ShapeDtypeStruct((_NC, _NPAD, _DIN), jnp.float32),
        jax.ShapeDtypeStruct((_NC, _NPAD, 8), jnp.float32),
    ],
    mesh=_mesh,
    compiler_params=pltpu.CompilerParams(use_tc_tiling_on_sc=False),
    scratch_types=[
        pltpu.VMEM((_CHA, _B), jnp.int32),
        pltpu.VMEM((_CHA, _B), jnp.int32),
        pltpu.VMEM((_B, _DIN), jnp.float32),
        pltpu.VMEM((_B, _DIN), jnp.float32),
        pltpu.VMEM((_B, _DIN), jnp.float32),
        pltpu.VMEM((_B, 8), jnp.float32),
        pltpu.VMEM_SHARED((_NPAD, _DIN), jnp.float32),
        pltpu.VMEM_SHARED((_NPAD, 8), jnp.float32),
        pltpu.SemaphoreType.DMA,
        pltpu.SemaphoreType.DMA,
        pltpu.SemaphoreType.DMA,
    ],
)
def _agg1c(x_hbm, src_hbm, dst_hbm, zd_hbm, zc_hbm, ones_hbm,
           p_out, c_out, src_v, dst_v, rows0, rows1, rows2, ones_v,
           acc_sh, cacc_sh, sem0, sem1, sem2):
    cid = lax.axis_index("c")
    sid = lax.axis_index("s")
    wid = cid * _NS + sid
    r0 = sid * _RPW
    pltpu.sync_copy(zd_hbm.at[pl.ds(r0, _RPW)], acc_sh.at[pl.ds(r0, _RPW)])
    pltpu.sync_copy(zc_hbm.at[pl.ds(r0, _RPW)], cacc_sh.at[pl.ds(r0, _RPW)])
    pltpu.sync_copy(ones_hbm, ones_v)
    plsc.subcore_barrier()

    rows = (rows0, rows1, rows2)
    sems = (sem0, sem1, sem2)

    def run_segment(n):
        pltpu.async_copy(x_hbm.at[src_v.at[0]], rows[0], sems[0])
        pltpu.async_copy(x_hbm.at[src_v.at[1]], rows[1], sems[1])

        def body(j, carry):
            i0 = 3 * j
            for b in range(3):
                nb = (b + 2) % 3
                pltpu.async_copy(
                    x_hbm.at[src_v.at[i0 + b + 2]], rows[nb], sems[nb])
                pltpu.sync_copy(ones_v, cacc_sh.at[dst_v.at[i0 + b]], add=True)
                pltpu.make_async_copy(
                    x_hbm.at[src_v.at[i0 + b]], rows[b], sems[b]).wait()
                pltpu.sync_copy(rows[b], acc_sh.at[dst_v.at[i0 + b]], add=True)
            return carry

        li = (n - 4) // 3 + 1
        lax.fori_loop(0, li, body, 0)
        for i in range(3 * li, n):
            if i + 2 < n:
                nb = (i + 2) % 3
                pltpu.async_copy(
                    x_hbm.at[src_v.at[i + 2]], rows[nb], sems[nb])
            pltpu.sync_copy(ones_v, cacc_sh.at[dst_v.at[i]], add=True)
            pltpu.make_async_copy(
                x_hbm.at[src_v.at[i]], rows[i % 3], sems[i % 3]).wait()
            pltpu.sync_copy(rows[i % 3], acc_sh.at[dst_v.at[i]], add=True)

    pltpu.sync_copy(src_hbm.at[wid, pl.ds(0, _CHA)], src_v)
    pltpu.sync_copy(dst_hbm.at[wid, pl.ds(0, _CHA)], dst_v)
    run_segment(_CHA)
    pltpu.sync_copy(src_hbm.at[wid, pl.ds(_CHA, _CHB)],
                    src_v.at[pl.ds(0, _CHB)])
    pltpu.sync_copy(dst_hbm.at[wid, pl.ds(_CHA, _CHB)],
                    dst_v.at[pl.ds(0, _CHB)])
    run_segment(_CHB)
    plsc.subcore_barrier()
    pltpu.sync_copy(acc_sh.at[pl.ds(r0, _RPW)], p_out.at[cid, pl.ds(r0, _RPW)])
    pltpu.sync_copy(cacc_sh.at[pl.ds(r0, _RPW)], c_out.at[cid, pl.ds(r0, _RPW)])


@functools.partial(
    pl.kernel,
    out_type=jax.ShapeDtypeStruct((_NC, _NPAD, _DOUT), jnp.float32),
    mesh=_mesh,
    compiler_params=pltpu.CompilerParams(use_tc_tiling_on_sc=False),
    scratch_types=[
        pltpu.VMEM((_CH, _B), jnp.int32),
        pltpu.VMEM((_CH, _B), jnp.int32),
        pltpu.VMEM((_B, _DOUT), jnp.float32),
        pltpu.VMEM((_B, _DOUT), jnp.float32),
        pltpu.VMEM((_B, _DOUT), jnp.float32),
        pltpu.VMEM((_B, _DOUT), jnp.float32),
        pltpu.VMEM_SHARED((_NPAD, _DOUT), jnp.float32),
        pltpu.SemaphoreType.DMA,
        pltpu.SemaphoreType.DMA,
        pltpu.SemaphoreType.DMA,
        pltpu.SemaphoreType.DMA,
    ],
)
def _agg2(x_hbm, src_hbm, dst_hbm, zd_hbm,
          p_out, src_v, dst_v, rows0, rows1, rows2, rows3, acc_sh,
          sem0, sem1, sem2, sem3):
    cid = lax.axis_index("c")
    sid = lax.axis_index("s")
    wid = cid * _NS + sid
    r0 = sid * _RPW
    pltpu.sync_copy(zd_hbm.at[pl.ds(r0, _RPW)], acc_sh.at[pl.ds(r0, _RPW)])
    pltpu.sync_copy(src_hbm.at[wid], src_v)
    pltpu.sync_copy(dst_hbm.at[wid], dst_v)
    plsc.subcore_barrier()

    rows = (rows0, rows1, rows2, rows3)
    sems = (sem0, sem1, sem2, sem3)

    for i in range(3):
        pltpu.async_copy(x_hbm.at[src_v.at[i]], rows[i], sems[i])

    def body(j, carry):
        i0 = 4 * j
        for b in range(4):
            nb = (b + 3) % 4
            pltpu.async_copy(x_hbm.at[src_v.at[i0 + b + 3]], rows[nb], sems[nb])
            pltpu.make_async_copy(
                x_hbm.at[src_v.at[i0 + b]], rows[b], sems[b]).wait()
            pltpu.sync_copy(rows[b], acc_sh.at[dst_v.at[i0 + b]], add=True)
        return carry

    lax.fori_loop(0, (_CH - 5) // 4, body, 0)
    for i in range(_CH - 5, _CH):
        b = i % 4
        if i + 3 < _CH:
            nb = (i + 3) % 4
            pltpu.async_copy(
                x_hbm.at[src_v.at[i + 3]], rows[nb], sems[nb])
        pltpu.make_async_copy(x_hbm.at[src_v.at[i]], rows[b], sems[b]).wait()
        pltpu.sync_copy(rows[b], acc_sh.at[dst_v.at[i]], add=True)
    plsc.subcore_barrier()
    pltpu.sync_copy(acc_sh.at[pl.ds(r0, _RPW)], p_out.at[cid, pl.ds(r0, _RPW)])


_BN = 1000


def _dense1_body(p_ref, c_ref, x_ref, wl_ref, b_ref, wr_ref, w2_ref,
                 h_ref, g_ref):
    cnt = jnp.sum(c_ref[...], axis=(0, 2))
    denom = jnp.maximum(cnt, 1.0)
    mean = (p_ref[0] + p_ref[1]) / denom[:, None]
    h = (jnp.dot(mean, wl_ref[...], preferred_element_type=jnp.float32)
         + b_ref[...]
         + jnp.dot(x_ref[...], wr_ref[...], preferred_element_type=jnp.float32))
    h = jnp.maximum(h, 0.0)
    h_ref[...] = h
    g_ref[...] = jnp.dot(h, w2_ref[...], preferred_element_type=jnp.float32)


def _dense1(P, C, x, WlT, b, WrT, W2T):
    return pl.pallas_call(
        _dense1_body,
        grid=(_N // _BN,),
        in_specs=[
            pl.BlockSpec((_NC, _BN, _DIN), lambda i: (0, i, 0)),
            pl.BlockSpec((_NC, _BN, 8), lambda i: (0, i, 0)),
            pl.BlockSpec((_BN, _DIN), lambda i: (i, 0)),
            pl.BlockSpec((_DIN, _DIN), lambda i: (0, 0)),
            pl.BlockSpec((1, _DIN), lambda i: (0, 0)),
            pl.BlockSpec((_DIN, _DIN), lambda i: (0, 0)),
            pl.BlockSpec((_DIN, _DOUT), lambda i: (0, 0)),
        ],
        out_specs=[
            pl.BlockSpec((_BN, _DIN), lambda i: (i, 0)),
            pl.BlockSpec((_BN, _DOUT), lambda i: (i, 0)),
        ],
        out_shape=[
            jax.ShapeDtypeStruct((_N, _DIN), jnp.float32),
            jax.ShapeDtypeStruct((_N, _DOUT), jnp.float32),
        ],
    )(P, C, x, WlT, b, WrT, W2T)


def _dense2_body(p_ref, c_ref, h_ref, w_ref, b_ref, o_ref):
    cnt = jnp.sum(c_ref[...], axis=(0, 2))
    denom = jnp.maximum(cnt, 1.0)
    mean = (p_ref[0] + p_ref[1]) / denom[:, None]
    o = (mean + b_ref[...]
         + jnp.dot(h_ref[...], w_ref[...], preferred_element_type=jnp.float32))
    m = jnp.max(o, axis=-1, keepdims=True)
    lse = jnp.log(jnp.sum(jnp.exp(o - m), axis=-1, keepdims=True)) + m
    o_ref[...] = o - lse


def _dense2(P, C, h, WrT, b):
    return pl.pallas_call(
        _dense2_body,
        grid=(_N // _BN,),
        in_specs=[
            pl.BlockSpec((_NC, _BN, _DOUT), lambda i: (0, i, 0)),
            pl.BlockSpec((_NC, _BN, 8), lambda i: (0, i, 0)),
            pl.BlockSpec((_BN, _DIN), lambda i: (i, 0)),
            pl.BlockSpec((_DIN, _DOUT), lambda i: (0, 0)),
            pl.BlockSpec((1, _DOUT), lambda i: (0, 0)),
        ],
        out_specs=pl.BlockSpec((_BN, _DOUT), lambda i: (i, 0)),
        out_shape=jax.ShapeDtypeStruct((_N, _DOUT), jnp.float32),
    )(P, C, h, WrT, b)


def kernel(x, edge_index, W1_l, b1_l, W1_r, W2_l, b2_l, W2_r):
    src = edge_index[0].reshape(_NW, _CH, _B)
    dst = edge_index[1].reshape(_NW, _CH, _B)
    zd = jnp.zeros((_NPAD, _DIN), jnp.float32)
    zc = jnp.zeros((_NPAD, 8), jnp.float32)
    z64 = jnp.zeros((_NPAD, _DOUT), jnp.float32)
    ones = jnp.zeros((_B, 8), jnp.float32).at[:, 0].set(1.0)
    P1, C1 = _agg1c(x, src, dst, zd, zc, ones)
    h, g = _dense1(P1, C1, x, W1_l.T, b1_l.reshape(1, -1), W1_r.T, W2_l.T)
    P2 = _agg2(g, src, dst, z64)
    return _dense2(P2, C1, h, W2_r.T, b2_l.reshape(1, -1))
